# baseline (speedup 1.0000x reference)
import functools

import jax
import jax.numpy as jnp
from jax.experimental import pallas as pl
from jax.experimental.pallas import tpu as pltpu

N = 512
M = 4
D = 16
HID = 32
OUT = 16
BU = 32
GRID = N // BU


def _body(eps_ref, g_ref, p_ref, w0_ref, b0_ref, w1_ref, b1_ref,
          rw0_ref, rb0_ref, rw1_ref, rb1_ref, out_ref,
          a_scr, x_scr):
    step = pl.program_id(0)
    u0 = step * BU

    blk = g_ref[...]
    x_scr[pl.ds(u0, BU), :] = blk[:, 0, :M * D]
    blk_t = jnp.swapaxes(blk, 1, 2)
    for c in range(M):
        a_scr[c, pl.ds(u0, BU), :N // 2] = jnp.sum(
            blk_t[:, 16 * c:16 * (c + 1), :], axis=1) * (1.0 / D)
        a_scr[c, pl.ds(u0, BU), N // 2:] = jnp.sum(
            blk_t[:, M * D + 16 * c:M * D + 16 * (c + 1), :], axis=1) * (1.0 / D)

    @pl.when(step == GRID - 1)
    def _phase2():
        s0 = 1.0 + eps_ref[0]
        s1 = 1.0 + eps_ref[1]
        perm = p_ref[...]
        w0 = w0_ref[...]
        b0 = b0_ref[...]
        w1 = w1_ref[...]
        b1 = b1_ref[...]
        x_all = x_scr[...]
        xp_all = jnp.dot(perm, x_all, preferred_element_type=jnp.float32)
        es = []
        for i in range(M):
            ai = a_scr[i]
            xi = x_all[:, 16 * i:16 * (i + 1)]
            xpi = xp_all[:, 16 * i:16 * (i + 1)]
            agg0 = jnp.dot(ai, xpi, preferred_element_type=jnp.float32)
            hp = jnp.maximum(jnp.dot(s0 * xi + agg0, w0,
                                     preferred_element_type=jnp.float32) + b0, 0.0)
            hm = jnp.maximum(jnp.dot(agg0 - s0 * xi, w0,
                                     preferred_element_type=jnp.float32) + b0, 0.0)
            h2 = jnp.concatenate([hp, hm], axis=1)
            h2p = jnp.dot(perm, h2, preferred_element_type=jnp.float32)
            agg1 = jnp.dot(ai, h2p, preferred_element_type=jnp.float32)
            ep = jnp.dot(s1 * hp + agg1[:, :HID], w1,
                         preferred_element_type=jnp.float32) + b1
            em = jnp.dot(s1 * hm - agg1[:, HID:], w1,
                         preferred_element_type=jnp.float32) + b1
            es.append(ep + em)
        xcat = jnp.concatenate(es, axis=1)
        hmid = jnp.maximum(jnp.dot(xcat, rw0_ref[...],
                                   preferred_element_type=jnp.float32)
                           + rb0_ref[...], 0.0)
        out_ref[...] = jnp.dot(hmid, rw1_ref[...],
                               preferred_element_type=jnp.float32) + rb1_ref[...]


def _full(shape):
    nd = len(shape)
    return pl.BlockSpec(shape, lambda i: (0,) * nd)


@jax.jit
def _run(g3, perm, enc_W0, enc_b0, enc_W1, enc_b1, enc_eps,
         rho_W0, rho_b0, rho_W1, rho_b1):
    return pl.pallas_call(
        _body,
        grid=(GRID,),
        in_specs=[
            pl.BlockSpec(memory_space=pltpu.SMEM),
            pl.BlockSpec((BU, N // 2, 2 * M * D), lambda i: (i, 0, 0)),
            _full((N, N)),
            _full((D, HID)), _full((1, HID)),
            _full((HID, OUT)), _full((1, OUT)),
            _full((M * OUT, HID)), _full((1, HID)),
            _full((HID, OUT)), _full((1, OUT)),
        ],
        out_specs=pl.BlockSpec((N, OUT), lambda i: (0, 0)),
        out_shape=jax.ShapeDtypeStruct((N, OUT), jnp.float32),
        scratch_shapes=[
            pltpu.VMEM((M, N, N), jnp.float32),
            pltpu.VMEM((N, M * D), jnp.float32),
        ],
        compiler_params=pltpu.CompilerParams(
            dimension_semantics=("arbitrary",),
        ),
    )(enc_eps, g3, perm, enc_W0, enc_b0, enc_W1, enc_b1,
      rho_W0, rho_b0, rho_W1, rho_b1)


def _perm_matrix():
    r = jax.lax.broadcasted_iota(jnp.int32, (N, N), 0)
    col = jax.lax.broadcasted_iota(jnp.int32, (N, N), 1)
    target = jnp.where(r < N // 2, 2 * r, 2 * r - (N - 1))
    return (col == target).astype(jnp.float32)


def kernel(g, enc_W0, enc_b0, enc_W1, enc_b1, enc_eps,
           rho_W0, rho_b0, rho_W1, rho_b1):
    g3 = g.reshape(N, N // 2, 2 * M * D)
    out = _run(g3, _perm_matrix(), enc_W0, enc_b0.reshape(1, HID),
               enc_W1, enc_b1.reshape(1, OUT), enc_eps,
               rho_W0, rho_b0.reshape(1, HID),
               rho_W1, rho_b1.reshape(1, OUT))
    return out[None]


# speedup vs baseline: 1.9036x; 1.9036x over previous
import functools

import jax
import jax.numpy as jnp
from jax import lax
from jax.experimental import pallas as pl
from jax.experimental.pallas import tpu as pltpu
from jax.experimental.pallas import tpu_sc as plsc

N = 512
M = 4
D = 16
W = 64
HID = 32
OUT = 16
NW = 32
CHUNK = 8 * N


UPW = N // NW


def _sc_reduce(g3):
    mesh = plsc.VectorSubcoreMesh(core_axis_name="c", subcore_axis_name="s")

    @functools.partial(
        pl.kernel,
        mesh=mesh,
        out_type=jax.ShapeDtypeStruct((N, M * N), jnp.float32),
        scratch_types=[
            pltpu.VMEM((1, W, N), jnp.float32),
            pltpu.VMEM((1, W, N), jnp.float32),
            pltpu.VMEM((1, M * N), jnp.float32),
            pltpu.SemaphoreType.DMA,
            pltpu.SemaphoreType.DMA,
            pltpu.SemaphoreType.DMA,
        ],
        compiler_params=pltpu.CompilerParams(
            use_tc_tiling_on_sc=True,
        ),
    )
    def k(g_hbm, a_hbm, buf_a, buf_b, arow, sem_a, sem_b, sem_out):
        cid = lax.axis_index("c")
        sid = lax.axis_index("s")
        wid = sid * 2 + cid
        u_base = wid * UPW

        def compute(buf, u):
            for c in range(M):
                def vg_body(vg, c2):
                    v0 = vg * 16
                    acc = buf[0, 16 * c, pl.ds(v0, 16)]
                    for kk in range(1, D):
                        acc = acc + buf[0, 16 * c + kk, pl.ds(v0, 16)]
                    arow[0, pl.ds(c * N + v0, 16)] = acc * (1.0 / D)
                    return c2
                lax.fori_loop(0, N // 16, vg_body, 0)
            pltpu.async_copy(arow, a_hbm.at[pl.ds(u, 1)], sem_out).wait()

        pltpu.async_copy(g_hbm.at[pl.ds(u_base, 1)], buf_a, sem_a)

        def pair_body(p, carry):
            u0 = u_base + 2 * p
            pltpu.async_copy(g_hbm.at[pl.ds(u0 + 1, 1)], buf_b, sem_b)
            pltpu.make_async_copy(g_hbm.at[pl.ds(u0, 1)], buf_a, sem_a).wait()
            compute(buf_a, u0)

            @pl.when(2 * p + 2 < UPW)
            def _():
                pltpu.async_copy(g_hbm.at[pl.ds(u0 + 2, 1)], buf_a, sem_a)

            pltpu.make_async_copy(g_hbm.at[pl.ds(u0 + 1, 1)], buf_b, sem_b).wait()
            compute(buf_b, u0 + 1)
            return carry

        lax.fori_loop(0, UPW // 2, pair_body, 0)

    return k(g3)


def _tc_body(eps_ref, a_ref, x_ref, w0_ref, b0_ref, w1_ref, b1_ref,
             rw0_ref, rb0_ref, rw1_ref, rb1_ref, out_ref):
    s0 = 1.0 + eps_ref[0]
    s1 = 1.0 + eps_ref[1]
    w0 = w0_ref[...]
    b0 = b0_ref[...]
    w1 = w1_ref[...]
    b1 = b1_ref[...]
    x_all = x_ref[...]
    es = []
    for i in range(M):
        ai = a_ref[:, N * i:N * (i + 1)]
        xi = x_all[:, 16 * i:16 * (i + 1)]
        agg0 = jnp.dot(ai, xi, preferred_element_type=jnp.float32)
        hp = jnp.maximum(jnp.dot(s0 * xi + agg0, w0,
                                 preferred_element_type=jnp.float32) + b0, 0.0)
        hm = jnp.maximum(jnp.dot(agg0 - s0 * xi, w0,
                                 preferred_element_type=jnp.float32) + b0, 0.0)
        h2 = jnp.concatenate([hp, hm], axis=1)
        agg1 = jnp.dot(ai, h2, preferred_element_type=jnp.float32)
        ep = jnp.dot(s1 * hp + agg1[:, :HID], w1,
                     preferred_element_type=jnp.float32) + b1
        em = jnp.dot(s1 * hm - agg1[:, HID:], w1,
                     preferred_element_type=jnp.float32) + b1
        es.append(ep + em)
    xcat = jnp.concatenate(es, axis=1)
    hmid = jnp.maximum(jnp.dot(xcat, rw0_ref[...],
                               preferred_element_type=jnp.float32)
                       + rb0_ref[...], 0.0)
    out_ref[...] = jnp.dot(hmid, rw1_ref[...],
                           preferred_element_type=jnp.float32) + rb1_ref[...]


def _full(shape):
    nd = len(shape)
    return pl.BlockSpec(shape, lambda: (0,) * nd)


def _tc_phase2(a2, x, enc_W0, enc_b0, enc_W1, enc_b1, enc_eps,
               rho_W0, rho_b0, rho_W1, rho_b1):
    return pl.pallas_call(
        _tc_body,
        in_specs=[
            pl.BlockSpec(memory_space=pltpu.SMEM),
            _full((N, M * N)),
            _full((N, W)),
            _full((D, HID)), _full((1, HID)),
            _full((HID, OUT)), _full((1, OUT)),
            _full((M * OUT, HID)), _full((1, HID)),
            _full((HID, OUT)), _full((1, OUT)),
        ],
        out_specs=pl.BlockSpec((N, OUT), lambda: (0, 0)),
        out_shape=jax.ShapeDtypeStruct((N, OUT), jnp.float32),
    )(enc_eps, a2, x, enc_W0, enc_b0, enc_W1, enc_b1,
      rho_W0, rho_b0, rho_W1, rho_b1)


@jax.jit
def _run(g2, x, enc_W0, enc_b0, enc_W1, enc_b1, enc_eps,
         rho_W0, rho_b0, rho_W1, rho_b1):
    a2 = _sc_reduce(g2)
    return _tc_phase2(a2, x, enc_W0, enc_b0, enc_W1, enc_b1, enc_eps,
                      rho_W0, rho_b0, rho_W1, rho_b1)


def kernel(g, enc_W0, enc_b0, enc_W1, enc_b1, enc_eps,
           rho_W0, rho_b0, rho_W1, rho_b1):
    g2 = jnp.transpose(g[0], (0, 2, 3, 1)).reshape(N, W, N)
    x = g[0, :, 0].reshape(N, W)
    out = _run(g2, x, enc_W0, enc_b0.reshape(1, HID),
               enc_W1, enc_b1.reshape(1, OUT), enc_eps,
               rho_W0, rho_b0.reshape(1, HID),
               rho_W1, rho_b1.reshape(1, OUT))
    return out[None]


# speedup vs baseline: 2.1825x; 1.1465x over previous
import functools

import jax
import jax.numpy as jnp
from jax import lax
from jax.experimental import pallas as pl
from jax.experimental.pallas import tpu as pltpu
from jax.experimental.pallas import tpu_sc as plsc

N = 512
M = 4
D = 16
W = 64
HID = 32
OUT = 16
NW = 32
S0 = 384
NSC = N - S0
UPW = NSC // NW
BU = 32
assert S0 % BU == 0 and NSC % (2 * NW) == 0


def _sc_reduce(g3):
    mesh = plsc.VectorSubcoreMesh(core_axis_name="c", subcore_axis_name="s")

    @functools.partial(
        pl.kernel,
        mesh=mesh,
        out_type=jax.ShapeDtypeStruct((NSC, M * N), jnp.float32),
        scratch_types=[
            pltpu.VMEM((1, W, N), jnp.float32),
            pltpu.VMEM((1, W, N), jnp.float32),
            pltpu.VMEM((1, M * N), jnp.float32),
            pltpu.SemaphoreType.DMA,
            pltpu.SemaphoreType.DMA,
            pltpu.SemaphoreType.DMA,
        ],
        compiler_params=pltpu.CompilerParams(
            use_tc_tiling_on_sc=True,
        ),
    )
    def k(g_hbm, a_hbm, buf_a, buf_b, arow, sem_a, sem_b, sem_out):
        cid = lax.axis_index("c")
        sid = lax.axis_index("s")
        wid = sid * 2 + cid
        u_base = S0 + wid * UPW

        def compute(buf, u):
            for c in range(M):
                def vg_body(vg, c2):
                    v0 = vg * 16
                    acc = buf[0, 16 * c, pl.ds(v0, 16)]
                    for kk in range(1, D):
                        acc = acc + buf[0, 16 * c + kk, pl.ds(v0, 16)]
                    arow[0, pl.ds(c * N + v0, 16)] = acc * (1.0 / D)
                    return c2
                lax.fori_loop(0, N // 16, vg_body, 0)
            pltpu.async_copy(arow, a_hbm.at[pl.ds(u - S0, 1)], sem_out).wait()

        pltpu.async_copy(g_hbm.at[pl.ds(u_base, 1)], buf_a, sem_a)

        def pair_body(p, carry):
            u0 = u_base + 2 * p
            pltpu.async_copy(g_hbm.at[pl.ds(u0 + 1, 1)], buf_b, sem_b)
            pltpu.make_async_copy(g_hbm.at[pl.ds(u0, 1)], buf_a, sem_a).wait()
            compute(buf_a, u0)

            @pl.when(2 * p + 2 < UPW)
            def _():
                pltpu.async_copy(g_hbm.at[pl.ds(u0 + 2, 1)], buf_a, sem_a)

            pltpu.make_async_copy(g_hbm.at[pl.ds(u0 + 1, 1)], buf_b, sem_b).wait()
            compute(buf_b, u0 + 1)
            return carry

        lax.fori_loop(0, UPW // 2, pair_body, 0)

    return k(g3)


def _tc_reduce_body(g_ref, out_ref):
    blk = g_ref[...]
    for c in range(M):
        out_ref[:, N * c:N * (c + 1)] = jnp.sum(
            blk[:, 16 * c:16 * (c + 1), :], axis=1) * (1.0 / D)


@jax.jit
def _tc_reduce(gt):
    return pl.pallas_call(
        _tc_reduce_body,
        grid=(S0 // BU,),
        in_specs=[pl.BlockSpec((BU, M * D, N), lambda i: (i, 0, 0))],
        out_specs=pl.BlockSpec((BU, M * N), lambda i: (i, 0)),
        out_shape=jax.ShapeDtypeStruct((S0, M * N), jnp.float32),
        compiler_params=pltpu.CompilerParams(
            dimension_semantics=("arbitrary",),
        ),
    )(gt)


def _tc_body(eps_ref, at_ref, ab_ref, x_ref, w0_ref, b0_ref, w1_ref, b1_ref,
             rw0_ref, rb0_ref, rw1_ref, rb1_ref, out_ref):
    s0 = 1.0 + eps_ref[0]
    s1 = 1.0 + eps_ref[1]
    w0 = w0_ref[...]
    b0 = b0_ref[...]
    w1 = w1_ref[...]
    b1 = b1_ref[...]
    x_all = x_ref[...]
    es = []
    for i in range(M):
        ai = jnp.concatenate(
            [at_ref[:, N * i:N * (i + 1)], ab_ref[:, N * i:N * (i + 1)]],
            axis=0)
        xi = x_all[:, 16 * i:16 * (i + 1)]
        agg0 = jnp.dot(ai, xi, preferred_element_type=jnp.float32)
        hp = jnp.maximum(jnp.dot(s0 * xi + agg0, w0,
                                 preferred_element_type=jnp.float32) + b0, 0.0)
        hm = jnp.maximum(jnp.dot(agg0 - s0 * xi, w0,
                                 preferred_element_type=jnp.float32) + b0, 0.0)
        h2 = jnp.concatenate([hp, hm], axis=1)
        agg1 = jnp.dot(ai, h2, preferred_element_type=jnp.float32)
        ep = jnp.dot(s1 * hp + agg1[:, :HID], w1,
                     preferred_element_type=jnp.float32) + b1
        em = jnp.dot(s1 * hm - agg1[:, HID:], w1,
                     preferred_element_type=jnp.float32) + b1
        es.append(ep + em)
    xcat = jnp.concatenate(es, axis=1)
    hmid = jnp.maximum(jnp.dot(xcat, rw0_ref[...],
                               preferred_element_type=jnp.float32)
                       + rb0_ref[...], 0.0)
    out_ref[...] = jnp.dot(hmid, rw1_ref[...],
                           preferred_element_type=jnp.float32) + rb1_ref[...]


def _full(shape):
    nd = len(shape)
    return pl.BlockSpec(shape, lambda: (0,) * nd)


def _tc_phase2(a_top, a_bot, x, enc_W0, enc_b0, enc_W1, enc_b1, enc_eps,
               rho_W0, rho_b0, rho_W1, rho_b1):
    return pl.pallas_call(
        _tc_body,
        in_specs=[
            pl.BlockSpec(memory_space=pltpu.SMEM),
            _full((S0, M * N)),
            _full((NSC, M * N)),
            _full((N, W)),
            _full((D, HID)), _full((1, HID)),
            _full((HID, OUT)), _full((1, OUT)),
            _full((M * OUT, HID)), _full((1, HID)),
            _full((HID, OUT)), _full((1, OUT)),
        ],
        out_specs=pl.BlockSpec((N, OUT), lambda: (0, 0)),
        out_shape=jax.ShapeDtypeStruct((N, OUT), jnp.float32),
    )(enc_eps, a_top, a_bot, x, enc_W0, enc_b0, enc_W1, enc_b1,
      rho_W0, rho_b0, rho_W1, rho_b1)


@jax.jit
def _run(gt, x, enc_W0, enc_b0, enc_W1, enc_b1, enc_eps,
         rho_W0, rho_b0, rho_W1, rho_b1):
    a_bot = _sc_reduce(gt)
    a_top = _tc_reduce(gt)
    return _tc_phase2(a_top, a_bot, x, enc_W0, enc_b0, enc_W1, enc_b1,
                      enc_eps, rho_W0, rho_b0, rho_W1, rho_b1)


def kernel(g, enc_W0, enc_b0, enc_W1, enc_b1, enc_eps,
           rho_W0, rho_b0, rho_W1, rho_b1):
    gt = jnp.transpose(g[0], (0, 2, 3, 1)).reshape(N, W, N)
    x = g[0, :, 0].reshape(N, W)
    out = _run(gt, x, enc_W0, enc_b0.reshape(1, HID),
               enc_W1, enc_b1.reshape(1, OUT), enc_eps,
               rho_W0, rho_b0.reshape(1, HID),
               rho_W1, rho_b1.reshape(1, OUT))
    return out[None]


# speedup vs baseline: 2.1845x; 1.0009x over previous
import functools

import jax
import jax.numpy as jnp
from jax import lax
from jax.experimental import pallas as pl
from jax.experimental.pallas import tpu as pltpu
from jax.experimental.pallas import tpu_sc as plsc

N = 512
M = 4
D = 16
W = 64
HID = 32
OUT = 16
NW = 32
S0 = 384
NSC = N - S0
UPW = NSC // NW
BU = 32
assert S0 % BU == 0 and NSC % (2 * NW) == 0


def _sc_reduce(g3):
    mesh = plsc.VectorSubcoreMesh(core_axis_name="c", subcore_axis_name="s")

    @functools.partial(
        pl.kernel,
        mesh=mesh,
        out_type=jax.ShapeDtypeStruct((NSC, M * N), jnp.float32),
        scratch_types=[
            pltpu.VMEM((4, 1, W // 2, N), jnp.float32),
            pltpu.VMEM((1, M * N), jnp.float32),
            pltpu.SemaphoreType.DMA,
            pltpu.SemaphoreType.DMA,
            pltpu.SemaphoreType.DMA,
            pltpu.SemaphoreType.DMA,
            pltpu.SemaphoreType.DMA,
        ],
        compiler_params=pltpu.CompilerParams(
            use_tc_tiling_on_sc=True,
        ),
    )
    def k(g_hbm, a_hbm, ring, arow, sem0, sem1, sem2, sem3, sem_out):
        cid = lax.axis_index("c")
        sid = lax.axis_index("s")
        wid = sid * 2 + cid
        u_base = S0 + wid * UPW
        sems = (sem0, sem1, sem2, sem3)
        nh = 2 * UPW

        def src(hh):
            return g_hbm.at[pl.ds(u_base + hh // 2, 1),
                            pl.ds((hh % 2) * (W // 2), W // 2)]

        def start(hh, bank):
            pltpu.async_copy(src(hh), ring.at[bank], sems[bank])

        def compute(bank, half, u):
            buf = ring.at[bank]
            for cc in range(2):
                c = 2 * half + cc
                def vg_body(vg, c2):
                    v0 = vg * 16
                    acc = buf[0, 16 * cc, pl.ds(v0, 16)]
                    for kk in range(1, D):
                        acc = acc + buf[0, 16 * cc + kk, pl.ds(v0, 16)]
                    arow[0, pl.ds(c * N + v0, 16)] = acc * (1.0 / D)
                    return c2
                lax.fori_loop(0, N // 16, vg_body, 0)
            if half == 1:
                pltpu.async_copy(
                    arow, a_hbm.at[pl.ds(u - S0, 1)], sem_out).wait()

        for hh in range(3):
            start(hh, hh)

        def quad_body(p, carry):
            for q in range(4):
                h0 = 4 * p + q
                @pl.when(h0 + 3 < nh)
                def _():
                    start(h0 + 3, (q + 3) % 4)
                pltpu.make_async_copy(src(h0), ring.at[q], sems[q]).wait()
                compute(q, q % 2, u_base + (4 * p + q) // 2)
            return carry

        lax.fori_loop(0, nh // 4, quad_body, 0)

    return k(g3)


def _tc_reduce_body(g_ref, out_ref):
    blk = g_ref[...]
    for c in range(M):
        out_ref[:, N * c:N * (c + 1)] = jnp.sum(
            blk[:, 16 * c:16 * (c + 1), :], axis=1) * (1.0 / D)


@jax.jit
def _tc_reduce(gt):
    return pl.pallas_call(
        _tc_reduce_body,
        grid=(S0 // BU,),
        in_specs=[pl.BlockSpec((BU, M * D, N), lambda i: (i, 0, 0))],
        out_specs=pl.BlockSpec((BU, M * N), lambda i: (i, 0)),
        out_shape=jax.ShapeDtypeStruct((S0, M * N), jnp.float32),
        compiler_params=pltpu.CompilerParams(
            dimension_semantics=("arbitrary",),
        ),
    )(gt)


def _tc_body(eps_ref, at_ref, ab_ref, x_ref, w0_ref, b0_ref, w1_ref, b1_ref,
             rw0_ref, rb0_ref, rw1_ref, rb1_ref, out_ref):
    s0 = 1.0 + eps_ref[0]
    s1 = 1.0 + eps_ref[1]
    w0 = w0_ref[...]
    b0 = b0_ref[...]
    w1 = w1_ref[...]
    b1 = b1_ref[...]
    x_all = x_ref[...]
    es = []
    for i in range(M):
        ai = jnp.concatenate(
            [at_ref[:, N * i:N * (i + 1)], ab_ref[:, N * i:N * (i + 1)]],
            axis=0)
        xi = x_all[:, 16 * i:16 * (i + 1)]
        agg0 = jnp.dot(ai, xi, preferred_element_type=jnp.float32)
        hp = jnp.maximum(jnp.dot(s0 * xi + agg0, w0,
                                 preferred_element_type=jnp.float32) + b0, 0.0)
        hm = jnp.maximum(jnp.dot(agg0 - s0 * xi, w0,
                                 preferred_element_type=jnp.float32) + b0, 0.0)
        h2 = jnp.concatenate([hp, hm], axis=1)
        agg1 = jnp.dot(ai, h2, preferred_element_type=jnp.float32)
        ep = jnp.dot(s1 * hp + agg1[:, :HID], w1,
                     preferred_element_type=jnp.float32) + b1
        em = jnp.dot(s1 * hm - agg1[:, HID:], w1,
                     preferred_element_type=jnp.float32) + b1
        es.append(ep + em)
    xcat = jnp.concatenate(es, axis=1)
    hmid = jnp.maximum(jnp.dot(xcat, rw0_ref[...],
                               preferred_element_type=jnp.float32)
                       + rb0_ref[...], 0.0)
    out_ref[...] = jnp.dot(hmid, rw1_ref[...],
                           preferred_element_type=jnp.float32) + rb1_ref[...]


def _full(shape):
    nd = len(shape)
    return pl.BlockSpec(shape, lambda: (0,) * nd)


def _tc_phase2(a_top, a_bot, x, enc_W0, enc_b0, enc_W1, enc_b1, enc_eps,
               rho_W0, rho_b0, rho_W1, rho_b1):
    return pl.pallas_call(
        _tc_body,
        in_specs=[
            pl.BlockSpec(memory_space=pltpu.SMEM),
            _full((S0, M * N)),
            _full((NSC, M * N)),
            _full((N, W)),
            _full((D, HID)), _full((1, HID)),
            _full((HID, OUT)), _full((1, OUT)),
            _full((M * OUT, HID)), _full((1, HID)),
            _full((HID, OUT)), _full((1, OUT)),
        ],
        out_specs=pl.BlockSpec((N, OUT), lambda: (0, 0)),
        out_shape=jax.ShapeDtypeStruct((N, OUT), jnp.float32),
    )(enc_eps, a_top, a_bot, x, enc_W0, enc_b0, enc_W1, enc_b1,
      rho_W0, rho_b0, rho_W1, rho_b1)


@jax.jit
def _run(gt, x, enc_W0, enc_b0, enc_W1, enc_b1, enc_eps,
         rho_W0, rho_b0, rho_W1, rho_b1):
    a_bot = _sc_reduce(gt)
    a_top = _tc_reduce(gt)
    return _tc_phase2(a_top, a_bot, x, enc_W0, enc_b0, enc_W1, enc_b1,
                      enc_eps, rho_W0, rho_b0, rho_W1, rho_b1)


def kernel(g, enc_W0, enc_b0, enc_W1, enc_b1, enc_eps,
           rho_W0, rho_b0, rho_W1, rho_b1):
    gt = jnp.transpose(g[0], (0, 2, 3, 1)).reshape(N, W, N)
    x = g[0, :, 0].reshape(N, W)
    out = _run(gt, x, enc_W0, enc_b0.reshape(1, HID),
               enc_W1, enc_b1.reshape(1, OUT), enc_eps,
               rho_W0, rho_b0.reshape(1, HID),
               rho_W1, rho_b1.reshape(1, OUT))
    return out[None]


# speedup vs baseline: 2.2450x; 1.0277x over previous
import functools

import jax
import jax.numpy as jnp
from jax import lax
from jax.experimental import pallas as pl
from jax.experimental.pallas import tpu as pltpu
from jax.experimental.pallas import tpu_sc as plsc

N = 512
M = 4
D = 16
W = 64
HID = 32
OUT = 16
NW = 32
S0 = 448
NSC = N - S0
UPW = NSC // NW
BU = 32
GRID = S0 // BU
assert S0 % BU == 0 and NSC % (2 * NW) == 0


def _sc_reduce(g3):
    mesh = plsc.VectorSubcoreMesh(core_axis_name="c", subcore_axis_name="s")

    @functools.partial(
        pl.kernel,
        mesh=mesh,
        out_type=jax.ShapeDtypeStruct((NSC, M * N), jnp.float32),
        scratch_types=[
            pltpu.VMEM((4, 1, W // 2, N), jnp.float32),
            pltpu.VMEM((1, M * N), jnp.float32),
            pltpu.SemaphoreType.DMA,
            pltpu.SemaphoreType.DMA,
            pltpu.SemaphoreType.DMA,
            pltpu.SemaphoreType.DMA,
            pltpu.SemaphoreType.DMA,
        ],
        compiler_params=pltpu.CompilerParams(
            use_tc_tiling_on_sc=True,
        ),
    )
    def k(g_hbm, a_hbm, ring, arow, sem0, sem1, sem2, sem3, sem_out):
        cid = lax.axis_index("c")
        sid = lax.axis_index("s")
        wid = sid * 2 + cid
        u_base = S0 + wid * UPW
        sems = (sem0, sem1, sem2, sem3)
        nh = 2 * UPW

        def src(hh):
            return g_hbm.at[pl.ds(u_base + hh // 2, 1),
                            pl.ds((hh % 2) * (W // 2), W // 2)]

        def start(hh, bank):
            pltpu.async_copy(src(hh), ring.at[bank], sems[bank])

        def compute(bank, half, u):
            buf = ring.at[bank]
            for cc in range(2):
                c = 2 * half + cc
                def vg_body(vg, c2):
                    v0 = vg * 16
                    acc = buf[0, 16 * cc, pl.ds(v0, 16)]
                    for kk in range(1, D):
                        acc = acc + buf[0, 16 * cc + kk, pl.ds(v0, 16)]
                    arow[0, pl.ds(c * N + v0, 16)] = acc * (1.0 / D)
                    return c2
                lax.fori_loop(0, N // 16, vg_body, 0)
            if half == 1:
                pltpu.async_copy(
                    arow, a_hbm.at[pl.ds(u - S0, 1)], sem_out).wait()

        for hh in range(3):
            start(hh, hh)

        def quad_body(p, carry):
            for q in range(4):
                h0 = 4 * p + q
                @pl.when(h0 + 3 < nh)
                def _():
                    start(h0 + 3, (q + 3) % 4)
                pltpu.make_async_copy(src(h0), ring.at[q], sems[q]).wait()
                compute(q, q % 2, u_base + (4 * p + q) // 2)
            return carry

        lax.fori_loop(0, nh // 4, quad_body, 0)

    return k(g3)


def _body(eps_ref, g_ref, ab_ref, x_ref, w0_ref, b0_ref, w1_ref, b1_ref,
          rw0_ref, rb0_ref, rw1_ref, rb1_ref, out_ref, a_scr):
    step = pl.program_id(0)
    u0 = step * BU

    blk = g_ref[...]
    for c in range(M):
        a_scr[c, pl.ds(u0, BU), :] = jnp.sum(
            blk[:, 16 * c:16 * (c + 1), :], axis=1) * (1.0 / D)

    @pl.when(step == GRID - 1)
    def _phase2():
        for c in range(M):
            a_scr[c, S0:, :] = ab_ref[:, N * c:N * (c + 1)]
        s0 = 1.0 + eps_ref[0]
        s1 = 1.0 + eps_ref[1]
        w0 = w0_ref[...]
        b0 = b0_ref[...]
        w1 = w1_ref[...]
        b1 = b1_ref[...]
        x_all = x_ref[...]
        es = []
        for i in range(M):
            ai = a_scr[i]
            xi = x_all[:, 16 * i:16 * (i + 1)]
            agg0 = jnp.dot(ai, xi, preferred_element_type=jnp.float32)
            hp = jnp.maximum(jnp.dot(s0 * xi + agg0, w0,
                                     preferred_element_type=jnp.float32) + b0, 0.0)
            hm = jnp.maximum(jnp.dot(agg0 - s0 * xi, w0,
                                     preferred_element_type=jnp.float32) + b0, 0.0)
            h2 = jnp.concatenate([hp, hm], axis=1)
            agg1 = jnp.dot(ai, h2, preferred_element_type=jnp.float32)
            ep = jnp.dot(s1 * hp + agg1[:, :HID], w1,
                         preferred_element_type=jnp.float32) + b1
            em = jnp.dot(s1 * hm - agg1[:, HID:], w1,
                         preferred_element_type=jnp.float32) + b1
            es.append(ep + em)
        xcat = jnp.concatenate(es, axis=1)
        hmid = jnp.maximum(jnp.dot(xcat, rw0_ref[...],
                                   preferred_element_type=jnp.float32)
                           + rb0_ref[...], 0.0)
        out_ref[...] = jnp.dot(hmid, rw1_ref[...],
                               preferred_element_type=jnp.float32) + rb1_ref[...]


def _full(shape):
    nd = len(shape)
    return pl.BlockSpec(shape, lambda i: (0,) * nd)


@jax.jit
def _run(gt, x, enc_W0, enc_b0, enc_W1, enc_b1, enc_eps,
         rho_W0, rho_b0, rho_W1, rho_b1):
    a_bot = _sc_reduce(gt)
    return pl.pallas_call(
        _body,
        grid=(GRID,),
        in_specs=[
            pl.BlockSpec(memory_space=pltpu.SMEM),
            pl.BlockSpec((BU, M * D, N), lambda i: (i, 0, 0)),
            _full((NSC, M * N)),
            _full((N, W)),
            _full((D, HID)), _full((1, HID)),
            _full((HID, OUT)), _full((1, OUT)),
            _full((M * OUT, HID)), _full((1, HID)),
            _full((HID, OUT)), _full((1, OUT)),
        ],
        out_specs=pl.BlockSpec((N, OUT), lambda i: (0, 0)),
        out_shape=jax.ShapeDtypeStruct((N, OUT), jnp.float32),
        scratch_shapes=[
            pltpu.VMEM((M, N, N), jnp.float32),
        ],
        compiler_params=pltpu.CompilerParams(
            dimension_semantics=("arbitrary",),
        ),
    )(enc_eps, gt, a_bot, x, enc_W0, enc_b0, enc_W1, enc_b1,
      rho_W0, rho_b0, rho_W1, rho_b1)


def kernel(g, enc_W0, enc_b0, enc_W1, enc_b1, enc_eps,
           rho_W0, rho_b0, rho_W1, rho_b1):
    gt = jnp.transpose(g[0], (0, 2, 3, 1)).reshape(N, W, N)
    x = g[0, :, 0].reshape(N, W)
    out = _run(gt, x, enc_W0, enc_b0.reshape(1, HID),
               enc_W1, enc_b1.reshape(1, OUT), enc_eps,
               rho_W0, rho_b0.reshape(1, HID),
               rho_W1, rho_b1.reshape(1, OUT))
    return out[None]
